# Initial kernel scaffold; baseline (speedup 1.0000x reference)
import jax
import jax.numpy as jnp
from jax.experimental import pallas as pl


def kernel(input, target):
    raise NotImplementedError("write your pallas kernel here")


# speedup vs baseline: 13.7706x; 13.7706x over previous
import functools

import jax
import jax.numpy as jnp
from jax import lax
from jax.experimental import pallas as pl
from jax.experimental.pallas import tpu as pltpu
from jax.experimental.pallas import tpu_sc as plsc

B = 4
C = 21
P = 512 * 512
K = 2048
NC = 2
NS = 16
L = 16
CH1 = 1024
CH2 = 2048
IMGS_PER_CORE = B // NC
PIX_PER_SUB = P // NS
TASKS_PER_CORE = IMGS_PER_CORE * C


def _sc_body(logits, labels, stats_out, inv_out,
             cbuf, invbuf, xbuf, ibuf, lbuf, hist, narr, farr, obuf):
    ci = lax.axis_index("c")
    s = lax.axis_index("s")
    iota = lax.iota(jnp.int32, L)
    fiota = iota.astype(jnp.float32)

    for bl in range(IMGS_PER_CORE):
        b = ci * IMGS_PER_CORE + bl
        sub_base = s * PIX_PER_SUB

        def chunk1(j, _, b=b, sub_base=sub_base):
            base = sub_base + j * CH1
            pltpu.sync_copy(logits.at[b, :, pl.ds(base, CH1)], cbuf)

            def vec1(i, _):
                acc = jnp.exp(cbuf[0, pl.ds(i * L, L)])
                for c in range(1, C):
                    acc = acc + jnp.exp(cbuf[c, pl.ds(i * L, L)])
                invbuf[pl.ds(i * L, L)] = 1.0 / acc
                return 0

            lax.fori_loop(0, CH1 // L, vec1, 0)
            pltpu.sync_copy(invbuf, inv_out.at[b, pl.ds(base, CH1)])
            return 0

        lax.fori_loop(0, PIX_PER_SUB // CH1, chunk1, 0)

    plsc.subcore_barrier()

    for slot in range((TASKS_PER_CORE + NS - 1) // NS):
        lt = s + slot * NS

        @pl.when(lt < TASKS_PER_CORE)
        def _():
            bl = lt // C
            c = lt - bl * C
            b = ci * IMGS_PER_CORE + bl
            row = b * C + c

            def zero(j, _):
                hist[pl.ds(j * L, L)] = jnp.zeros((L,), jnp.float32)
                return 0

            lax.fori_loop(0, (2 * L * K) // L, zero, 0)

            def chunk2(j, _):
                base = j * CH2
                pltpu.sync_copy(logits.at[b, c, pl.ds(base, CH2)], xbuf)
                pltpu.sync_copy(inv_out.at[b, pl.ds(base, CH2)], ibuf)
                pltpu.sync_copy(labels.at[b, pl.ds(base, CH2)], lbuf)

                def vec2(i, _):
                    x = xbuf[pl.ds(i * L, L)]
                    inv = ibuf[pl.ds(i * L, L)]
                    lab = lbuf[pl.ds(i * L, L)]
                    p = jnp.exp(x) * inv
                    fg = lab == c
                    e = jnp.where(fg, 1.0 - p, p)
                    kb = (e * float(K)).astype(jnp.int32)
                    kb = jnp.minimum(jnp.maximum(kb, 0), K - 1)
                    idx = iota * K + (K - 1 - kb)
                    plsc.addupdate_scatter(hist, [idx],
                                           jnp.ones((L,), jnp.float32))
                    plsc.addupdate_scatter(hist, [idx + L * K],
                                           jnp.where(fg, 1.0, 0.0))
                    return 0

                lax.fori_loop(0, CH2 // L, vec2, 0)
                return 0

            lax.fori_loop(0, P // CH2, chunk2, 0)

            def red(j, carry):
                cn, cf = carry
                vc = hist[pl.ds(j * L, L)]
                vf = hist[pl.ds(L * K + j * L, L)]
                for l in range(1, L):
                    vc = vc + hist[pl.ds(l * K + j * L, L)]
                    vf = vf + hist[pl.ds(L * K + l * K + j * L, L)]
                nv = plsc.cumsum(vc) + cn
                fv = plsc.cumsum(vf) + cf
                narr[pl.ds(j * L, L)] = nv
                farr[pl.ds(j * L, L)] = fv
                return (cn + jnp.sum(vc), cf + jnp.sum(vf))

            _, g = lax.fori_loop(0, K // L, red, (0.0, 0.0))

            def jacc(j, a):
                nv = narr[pl.ds(j * L, L)]
                fv = farr[pl.ds(j * L, L)]
                jv = nv / jnp.maximum(g + nv - fv, 1.0)
                return a + jnp.sum(jv)

            acc = lax.fori_loop(0, K // L, jacc, 0.0)

            nl = narr[pl.ds(K - L, L)]
            fl = farr[pl.ds(K - L, L)]
            jl = nl / jnp.maximum(g + nl - fl, 1.0)
            jlast = jnp.sum(jnp.where(iota == L - 1, jl, 0.0))
            loss = (acc - 0.5 * jlast) * (1.0 / K)

            present = jnp.where(g > 0.0, 1.0, 0.0)
            ov = jnp.where(iota == 0, loss * present,
                           jnp.where(iota == 1, present, 0.0))
            obuf[...] = ov
            pltpu.sync_copy(obuf, stats_out.at[row])


@jax.jit
def _sc_call(logits, labels):
    mesh = plsc.VectorSubcoreMesh(core_axis_name="c", subcore_axis_name="s")
    f = pl.kernel(
        _sc_body,
        out_type=[
            jax.ShapeDtypeStruct((B * C, L), jnp.float32),
            jax.ShapeDtypeStruct((B, P), jnp.float32),
        ],
        mesh=mesh,
        compiler_params=pltpu.CompilerParams(needs_layout_passes=False),
        scratch_types=[
            pltpu.VMEM((C, CH1), jnp.float32),
            pltpu.VMEM((CH1,), jnp.float32),
            pltpu.VMEM((CH2,), jnp.float32),
            pltpu.VMEM((CH2,), jnp.float32),
            pltpu.VMEM((CH2,), jnp.int32),
            pltpu.VMEM((2 * L * K,), jnp.float32),
            pltpu.VMEM((K,), jnp.float32),
            pltpu.VMEM((K,), jnp.float32),
            pltpu.VMEM((L,), jnp.float32),
        ],
    )
    return f(logits, labels)


def _combine_body(stats_ref, o_ref):
    st = stats_ref[...]
    row = lax.broadcasted_iota(jnp.int32, (B * C, L), 0)
    col = lax.broadcasted_iota(jnp.int32, (B * C, L), 1)
    img = row // C
    total = 0.0
    for b in range(B):
        sel = img == b
        numer = jnp.sum(jnp.where(sel & (col == 0), st, 0.0))
        denom = jnp.sum(jnp.where(sel & (col == 1), st, 0.0))
        total = total + numer / jnp.maximum(denom, 1.0)
    o_ref[0, 0] = total / float(B)


@jax.jit
def _combine(stats):
    return pl.pallas_call(
        _combine_body,
        out_shape=jax.ShapeDtypeStruct((1, 1), jnp.float32),
        out_specs=pl.BlockSpec(memory_space=pltpu.SMEM),
    )(stats)


def kernel(input, target):
    logits = input.reshape(B, C, P)
    labels = target.reshape(B, P).astype(jnp.int32)
    stats, _ = _sc_call(logits, labels)
    out = _combine(stats)
    return out.reshape(())


# speedup vs baseline: 62.0635x; 4.5070x over previous
import functools

import jax
import jax.numpy as jnp
from jax import lax
from jax.experimental import pallas as pl
from jax.experimental.pallas import tpu as pltpu
from jax.experimental.pallas import tpu_sc as plsc

B = 4
C = 21
P = 512 * 512
K = 2048
NC = 2
NS = 16
L = 16
CH1 = 512
CH2 = 2048
IMGS_PER_CORE = B // NC
PIX_PER_SUB = P // NS
TASKS_PER_CORE = IMGS_PER_CORE * C
NCH1 = PIX_PER_SUB // CH1
NCH2 = P // CH2


def _sc_body(logits, labels, stats_out, inv_out,
             cb0, cb1, invbuf, xb0, xb1, ib0, ib1, lb0, lb1,
             hist, narr, farr, obuf, s1a, s1b, s2a, s2b):
    ci = lax.axis_index("c")
    s = lax.axis_index("s")
    iota = lax.iota(jnp.int32, L)

    for bl in range(IMGS_PER_CORE):
        b = ci * IMGS_PER_CORE + bl
        sub_base = s * PIX_PER_SUB

        def issue1(j, cb, sem, b=b, sub_base=sub_base):
            pltpu.make_async_copy(
                logits.at[b, :, pl.ds(sub_base + j * CH1, CH1)], cb, sem
            ).start()

        def wait1(cb, sem, b=b, sub_base=sub_base):
            pltpu.make_async_copy(
                logits.at[b, :, pl.ds(sub_base, CH1)], cb, sem
            ).wait()

        def compute1(j, cb, b=b, sub_base=sub_base):
            @plsc.parallel_loop(0, CH1 // L, unroll=2)
            def vec1(i):
                acc = jnp.exp(cb[0, pl.ds(i * L, L)])
                for c in range(1, C):
                    acc = acc + jnp.exp(cb[c, pl.ds(i * L, L)])
                invbuf[pl.ds(i * L, L)] = 1.0 / acc
            pltpu.sync_copy(
                invbuf, inv_out.at[b, pl.ds(sub_base + j * CH1, CH1)]
            )

        issue1(0, cb0, s1a)

        def pair1(j2, _):
            j = 2 * j2
            issue1(j + 1, cb1, s1b)
            wait1(cb0, s1a)
            compute1(j, cb0)

            @pl.when(j2 < NCH1 // 2 - 1)
            def _():
                issue1(j + 2, cb0, s1a)

            wait1(cb1, s1b)
            compute1(j + 1, cb1)
            return 0

        lax.fori_loop(0, NCH1 // 2, pair1, 0)

    plsc.subcore_barrier()

    for slot in range((TASKS_PER_CORE + NS - 1) // NS):
        lt = s + slot * NS

        @pl.when(lt < TASKS_PER_CORE)
        def _():
            bl = lt // C
            c = lt - bl * C
            b = ci * IMGS_PER_CORE + bl
            row = b * C + c

            @plsc.parallel_loop(0, (2 * L * K) // L, unroll=8)
            def zero(j):
                hist[pl.ds(j * L, L)] = jnp.zeros((L,), jnp.float32)

            def issue2(j, xb, ib, lb, sem, b=b, c=c):
                base = j * CH2
                pltpu.make_async_copy(
                    logits.at[b, c, pl.ds(base, CH2)], xb, sem).start()
                pltpu.make_async_copy(
                    inv_out.at[b, pl.ds(base, CH2)], ib, sem).start()
                pltpu.make_async_copy(
                    labels.at[b, pl.ds(base, CH2)], lb, sem).start()

            def wait2(xb, ib, lb, sem, b=b, c=c):
                pltpu.make_async_copy(
                    logits.at[b, c, pl.ds(0, CH2)], xb, sem).wait()
                pltpu.make_async_copy(
                    inv_out.at[b, pl.ds(0, CH2)], ib, sem).wait()
                pltpu.make_async_copy(
                    labels.at[b, pl.ds(0, CH2)], lb, sem).wait()

            def compute2(xb, ib, lb, c=c):
                @plsc.parallel_loop(0, CH2 // L, unroll=4)
                def vec2(i):
                    x = xb[pl.ds(i * L, L)]
                    inv = ib[pl.ds(i * L, L)]
                    lab = lb[pl.ds(i * L, L)]
                    p = jnp.exp(x) * inv
                    fg = lab == c
                    e = jnp.where(fg, 1.0 - p, p)
                    kb = jnp.minimum(e * float(K), float(K - 1))
                    kb = kb.astype(jnp.int32)
                    idx = iota * K + (K - 1 - kb)
                    plsc.addupdate_scatter(hist, [idx],
                                           jnp.ones((L,), jnp.float32))
                    plsc.addupdate_scatter(hist, [idx + L * K],
                                           jnp.where(fg, 1.0, 0.0))

            issue2(0, xb0, ib0, lb0, s2a)

            def pair2(j2, _):
                j = 2 * j2
                issue2(j + 1, xb1, ib1, lb1, s2b)
                wait2(xb0, ib0, lb0, s2a)
                compute2(xb0, ib0, lb0)

                @pl.when(j2 < NCH2 // 2 - 1)
                def _():
                    issue2(j + 2, xb0, ib0, lb0, s2a)

                wait2(xb1, ib1, lb1, s2b)
                compute2(xb1, ib1, lb1)
                return 0

            lax.fori_loop(0, NCH2 // 2, pair2, 0)

            def red(j, carry):
                cn, cf = carry
                vc = hist[pl.ds(j * L, L)]
                vf = hist[pl.ds(L * K + j * L, L)]
                for l in range(1, L):
                    vc = vc + hist[pl.ds(l * K + j * L, L)]
                    vf = vf + hist[pl.ds(L * K + l * K + j * L, L)]
                nv = plsc.cumsum(vc) + cn
                fv = plsc.cumsum(vf) + cf
                narr[pl.ds(j * L, L)] = nv
                farr[pl.ds(j * L, L)] = fv
                return (cn + jnp.sum(vc), cf + jnp.sum(vf))

            _, g = plsc.parallel_loop(
                0, K // L, unroll=2,
                carry=(jnp.float32(0.0), jnp.float32(0.0)))(red)

            def jacc(j, a):
                nv = narr[pl.ds(j * L, L)]
                fv = farr[pl.ds(j * L, L)]
                jv = nv / jnp.maximum(g + nv - fv, 1.0)
                return a + jnp.sum(jv)

            acc = plsc.parallel_loop(
                0, K // L, unroll=2, carry=jnp.float32(0.0))(jacc)

            nl = narr[pl.ds(K - L, L)]
            fl = farr[pl.ds(K - L, L)]
            jl = nl / jnp.maximum(g + nl - fl, 1.0)
            jlast = jnp.sum(jnp.where(iota == L - 1, jl, 0.0))
            loss = (acc - 0.5 * jlast) * (1.0 / K)

            present = jnp.where(g > 0.0, 1.0, 0.0)
            ov = jnp.where(iota == 0, loss * present,
                           jnp.where(iota == 1, present, 0.0))
            obuf[...] = ov
            pltpu.sync_copy(obuf, stats_out.at[row])


@jax.jit
def _sc_call(logits, labels):
    mesh = plsc.VectorSubcoreMesh(core_axis_name="c", subcore_axis_name="s")
    f = pl.kernel(
        _sc_body,
        out_type=[
            jax.ShapeDtypeStruct((B * C, L), jnp.float32),
            jax.ShapeDtypeStruct((B, P), jnp.float32),
        ],
        mesh=mesh,
        compiler_params=pltpu.CompilerParams(needs_layout_passes=False),
        scratch_types=[
            pltpu.VMEM((C, CH1), jnp.float32),
            pltpu.VMEM((C, CH1), jnp.float32),
            pltpu.VMEM((CH1,), jnp.float32),
            pltpu.VMEM((CH2,), jnp.float32),
            pltpu.VMEM((CH2,), jnp.float32),
            pltpu.VMEM((CH2,), jnp.float32),
            pltpu.VMEM((CH2,), jnp.float32),
            pltpu.VMEM((CH2,), jnp.int32),
            pltpu.VMEM((CH2,), jnp.int32),
            pltpu.VMEM((2 * L * K,), jnp.float32),
            pltpu.VMEM((K,), jnp.float32),
            pltpu.VMEM((K,), jnp.float32),
            pltpu.VMEM((L,), jnp.float32),
            pltpu.SemaphoreType.DMA,
            pltpu.SemaphoreType.DMA,
            pltpu.SemaphoreType.DMA,
            pltpu.SemaphoreType.DMA,
        ],
    )
    return f(logits, labels)


def _combine_body(stats_ref, o_ref):
    st = stats_ref[...]
    row = lax.broadcasted_iota(jnp.int32, (B * C, L), 0)
    col = lax.broadcasted_iota(jnp.int32, (B * C, L), 1)
    img = row // C
    total = 0.0
    for b in range(B):
        sel = img == b
        numer = jnp.sum(jnp.where(sel & (col == 0), st, 0.0))
        denom = jnp.sum(jnp.where(sel & (col == 1), st, 0.0))
        total = total + numer / jnp.maximum(denom, 1.0)
    o_ref[0, 0] = total / float(B)


@jax.jit
def _combine(stats):
    return pl.pallas_call(
        _combine_body,
        out_shape=jax.ShapeDtypeStruct((1, 1), jnp.float32),
        out_specs=pl.BlockSpec(memory_space=pltpu.SMEM),
    )(stats)


def kernel(input, target):
    logits = input.reshape(B, C, P)
    labels = target.reshape(B, P).astype(jnp.int32)
    stats, _ = _sc_call(logits, labels)
    out = _combine(stats)
    return out.reshape(())
